# baseline (speedup 1.0000x reference)
import functools

import jax
import jax.numpy as jnp
from jax import lax
from jax.experimental import pallas as pl
from jax.experimental.pallas import tpu as pltpu
from jax.experimental.pallas import tpu_sc as plsc

_N = 10000
_NPAD = 10240
_E = 320000
_D = 128
_NC = 2
_NS = 16
_NW = _NC * _NS
_EPW = 10112
_EPAD = _EPW * _NW
_G = 16
_NG = _EPW // _G
_NSLOT = 2
_NGH = _NG // (2 * _NSLOT)
_CH = _NPAD // _NS
_NAGG = 10112
_CHA = _NAGG // _NS
_PB = 512


def _pre_body(x_ref, w_ref, b_ref, wm_ref, bm_ref, wa_ref,
              h_ref, g_ref, es1_ref, es2_ref):
    x = x_ref[...]
    h_raw = lax.dot_general(x, w_ref[...], (((1,), (1,)), ((), ())),
                            preferred_element_type=jnp.float32) + b_ref[...]
    nrm = jnp.sqrt(jnp.sum(h_raw * h_raw, axis=1, keepdims=True))
    h_dir = h_raw / jnp.maximum(nrm, 1e-12)
    mag_lin = jnp.sum(x * wm_ref[...], axis=1, keepdims=True) + bm_ref[...]
    mag = jax.nn.sigmoid(mag_lin) * 1.5
    h = h_dir * mag
    h_ref[...] = h
    wa = wa_ref[...]
    es1 = jnp.exp(jnp.sum(h * wa[0:1, :], axis=1, keepdims=True))
    es1_ref[...] = es1
    es2_ref[...] = jnp.exp(jnp.sum(h * wa[1:2, :], axis=1, keepdims=True))
    g_ref[...] = h * es1


_pre_call = pl.pallas_call(
    _pre_body,
    grid=(_NPAD // _PB,),
    in_specs=[
        pl.BlockSpec((_PB, _D), lambda i: (i, 0)),
        pl.BlockSpec((_D, _D), lambda i: (0, 0)),
        pl.BlockSpec((1, _D), lambda i: (0, 0)),
        pl.BlockSpec((1, _D), lambda i: (0, 0)),
        pl.BlockSpec((1, 1), lambda i: (0, 0)),
        pl.BlockSpec((2, _D), lambda i: (0, 0)),
    ],
    out_specs=[
        pl.BlockSpec((_PB, _D), lambda i: (i, 0)),
        pl.BlockSpec((_PB, _D), lambda i: (i, 0)),
        pl.BlockSpec((_PB, 1), lambda i: (i, 0)),
        pl.BlockSpec((_PB, 1), lambda i: (i, 0)),
    ],
    out_shape=[
        jax.ShapeDtypeStruct((_NPAD, _D), jnp.float32),
        jax.ShapeDtypeStruct((_NPAD, _D), jnp.float32),
        jax.ShapeDtypeStruct((_NPAD, 1), jnp.float32),
        jax.ShapeDtypeStruct((_NPAD, 1), jnp.float32),
    ],
)


def _post_body(h_ref, a0_ref, a1_ref, es2_ref, s10_ref, s11_ref, o_ref):
    es2 = es2_ref[...]
    f = es2 / (es2 * (s10_ref[...] + s11_ref[...]) + 1e-10)
    hc = h_ref[...] + f * (a0_ref[...] + a1_ref[...])
    un = jnp.sqrt(jnp.sum(hc * hc, axis=1, keepdims=True))
    unc = jnp.maximum(un, 1e-15)
    hp = jnp.tanh(unc) * hc / unc
    nn = jnp.sqrt(jnp.sum(hp * hp, axis=1, keepdims=True))
    o_ref[...] = jnp.where(nn > 0.95, hp * (0.95 / (nn + 1e-8)), hp)


_post_call = pl.pallas_call(
    _post_body,
    grid=(_NPAD // _PB,),
    in_specs=[pl.BlockSpec((_PB, _D), lambda i: (i, 0))] * 3
    + [pl.BlockSpec((_PB, 1), lambda i: (i, 0))] * 3,
    out_specs=pl.BlockSpec((_PB, _D), lambda i: (i, 0)),
    out_shape=jax.ShapeDtypeStruct((_NPAD, _D), jnp.float32),
)


_mesh = plsc.VectorSubcoreMesh(core_axis_name="c", subcore_axis_name="s")


@functools.partial(
    pl.kernel,
    out_type=jax.ShapeDtypeStruct((_NC, _NPAD), jnp.float32),
    mesh=_mesh,
    compiler_params=pltpu.CompilerParams(needs_layout_passes=False),
    scratch_types=[
        pltpu.VMEM((_NPAD,), jnp.float32),
        pltpu.VMEM((_EPW,), jnp.int32),
        pltpu.VMEM((_EPW,), jnp.int32),
        pltpu.VMEM((_NPAD,), jnp.float32),
        pltpu.VMEM((_NS, _CH), jnp.float32),
        pltpu.VMEM_SHARED((_NS, _NPAD), jnp.float32),
    ],
)
def _edge_phase_a(src_h, tgt_h, es1_h, sump_h,
                  es1_v, src_v, tgt_v, sum_v, red_v, shared):
    c = lax.axis_index("c")
    s = lax.axis_index("s")
    wid = s * _NC + c
    base = wid * _EPW
    pltpu.sync_copy(es1_h, es1_v)
    pltpu.sync_copy(src_h.at[pl.ds(base, _EPW)], src_v)
    pltpu.sync_copy(tgt_h.at[pl.ds(base, _EPW)], tgt_v)

    zv = jnp.zeros((16,), jnp.float32)

    def _zero(i, carry):
        sum_v[pl.ds(i * 16, 16)] = zv
        return carry

    lax.fori_loop(0, _NPAD // 16, _zero, 0)

    def _edges(i, carry):
        o = i * 16
        si = src_v[pl.ds(o, 16)]
        ti = tgt_v[pl.ds(o, 16)]
        plsc.addupdate_scatter(sum_v, [ti], plsc.load_gather(es1_v, [si]))
        return carry

    lax.fori_loop(0, _EPW // 16, _edges, 0)

    pltpu.sync_copy(sum_v, shared.at[s])
    plsc.subcore_barrier()

    ch = s * _CH
    pltpu.sync_copy(shared.at[:, pl.ds(ch, _CH)], red_v)

    def _reduce(j, carry):
        acc = red_v[0, pl.ds(j * 16, 16)]
        for k in range(1, _NS):
            acc = acc + red_v[k, pl.ds(j * 16, 16)]
        sum_v[pl.ds(j * 16, 16)] = acc
        return carry

    lax.fori_loop(0, _CH // 16, _reduce, 0)
    pltpu.sync_copy(sum_v.at[pl.ds(0, _CH)], sump_h.at[c, pl.ds(ch, _CH)])


@functools.partial(
    pl.kernel,
    out_type=jax.ShapeDtypeStruct((_NC, _NAGG, _D), jnp.float32),
    mesh=_mesh,
    compiler_params=pltpu.CompilerParams(needs_layout_passes=False),
    scratch_types=[
        pltpu.VMEM((_EPW,), jnp.int32),
        pltpu.VMEM((_EPW,), jnp.int32),
    ]
    + [pltpu.VMEM((_G, _D), jnp.float32)] * (2 * _NSLOT)
    + [pltpu.VMEM_SHARED((_NAGG, _D), jnp.float32)]
    + [pltpu.SemaphoreType.DMA] * (2 * _NSLOT),
)
def _edge_phase_b(src_h, tgt_h, g_hbm, aggp_h,
                  src_v, tgt_v, *rest):
    bufs = rest[:2 * _NSLOT]
    agg_s = rest[2 * _NSLOT]
    sems = rest[2 * _NSLOT + 1:]
    gsem = sems[:_NSLOT]
    ssem = sems[_NSLOT:]
    c = lax.axis_index("c")
    s = lax.axis_index("s")
    wid = s * _NC + c
    base = wid * _EPW
    pltpu.sync_copy(src_h.at[pl.ds(base, _EPW)], src_v)
    pltpu.sync_copy(tgt_h.at[pl.ds(base, _EPW)], tgt_v)

    zv = jnp.zeros((16,), jnp.float32)
    for b in range(_NSLOT):
        for j in range(_G):
            for k in range(_D // 16):
                bufs[2 * b + 1][j, pl.ds(k * 16, 16)] = zv

    rbase = s * _CHA
    _ZF = _CHA // _G
    _ZR = _CHA - _ZF * _G

    def _zero(k, carry):
        pltpu.sync_copy(bufs[1], agg_s.at[pl.ds(rbase + k * _G, _G), :])
        return carry

    lax.fori_loop(0, _ZF, _zero, 0)
    pltpu.sync_copy(bufs[1].at[pl.ds(0, _ZR), :],
                    agg_s.at[pl.ds(rbase + _ZF * _G, _ZR), :])
    plsc.subcore_barrier()

    dummy = jnp.full((16,), _N, jnp.int32)
    for b in range(_NSLOT):
        si = src_v[pl.ds(b * _G, _G)]
        pltpu.async_copy(g_hbm.at[si], bufs[2 * b], gsem[b])
        pltpu.async_copy(bufs[2 * b + 1], agg_s.at[dummy], ssem[b],
                         add=True)

    def _double_turn(k2, carry):
        for phase in (0, 1):
            cur = phase
            nxt = 1 - phase
            for b in range(_NSLOT):
                g = (k2 * 2 + phase) * _NSLOT + b
                o = g * _G
                siw = src_v[pl.ds(o, _G)]
                pltpu.make_async_copy(g_hbm.at[siw], bufs[2 * b + cur],
                                      gsem[b]).wait()

                @pl.when(g + _NSLOT < _NG)
                def _next_gather():
                    pltpu.make_async_copy(bufs[2 * b + nxt],
                                          agg_s.at[dummy], ssem[b]).wait()
                    si = src_v[pl.ds(o + _NSLOT * _G, _G)]
                    pltpu.async_copy(g_hbm.at[si], bufs[2 * b + nxt],
                                     gsem[b])

                ti = tgt_v[pl.ds(o, _G)]
                pltpu.async_copy(bufs[2 * b + cur], agg_s.at[ti], ssem[b],
                                 add=True)
        return carry

    lax.fori_loop(0, _NGH, _double_turn, 0)

    for b in range(_NSLOT):
        pltpu.make_async_copy(bufs[2 * b], agg_s.at[dummy],
                              ssem[b]).wait()
        pltpu.make_async_copy(bufs[2 * b + 1], agg_s.at[dummy],
                              ssem[b]).wait()
    plsc.subcore_barrier()

    def _out(k, carry):
        r = rbase + k * _G
        pltpu.sync_copy(agg_s.at[pl.ds(r, _G), :], bufs[0])
        pltpu.sync_copy(bufs[0], aggp_h.at[c, pl.ds(r, _G), :])
        return carry

    lax.fori_loop(0, _ZF, _out, 0)
    r_tail = rbase + _ZF * _G
    pltpu.sync_copy(agg_s.at[pl.ds(r_tail, _ZR), :],
                    bufs[0].at[pl.ds(0, _ZR), :])
    pltpu.sync_copy(bufs[0].at[pl.ds(0, _ZR), :],
                    aggp_h.at[c, pl.ds(r_tail, _ZR), :])


def kernel(x, edge_index, W, b, Wm, bm, Wa):
    x_p = jnp.pad(x, ((0, _NPAD - _N), (0, 0)))
    h, g, es1c, es2c = _pre_call(x_p, W, b.reshape(1, _D), Wm,
                                 bm.reshape(1, 1), Wa.reshape(2, _D))
    pad_e = _EPAD - _E
    src = jnp.concatenate([edge_index[0], jnp.zeros((pad_e,), jnp.int32)])
    tgt = jnp.concatenate([edge_index[1], jnp.full((pad_e,), _N, jnp.int32)])
    es1 = es1c.reshape(_NPAD)
    sump = _edge_phase_a(src, tgt, es1)
    aggp = _edge_phase_b(src, tgt, g)
    agg0 = jnp.pad(aggp[0], ((0, _NPAD - _NAGG), (0, 0)))
    agg1 = jnp.pad(aggp[1], ((0, _NPAD - _NAGG), (0, 0)))
    s10 = sump[0].reshape(_NPAD, 1)
    s11 = sump[1].reshape(_NPAD, 1)
    out = _post_call(h, agg0, agg1, es2c, s10, s11)
    return out[:_N]


# speedup vs baseline: 1.3040x; 1.3040x over previous
import functools

import jax
import jax.numpy as jnp
from jax import lax
from jax.experimental import pallas as pl
from jax.experimental.pallas import tpu as pltpu
from jax.experimental.pallas import tpu_sc as plsc

_N = 10000
_NPAD = 10240
_E = 320000
_D = 128
_NC = 2
_NS = 16
_NW = _NC * _NS
_EPW = 10080
_EPAD = _EPW * _NW
_G = 16
_NG = _EPW // _G
_NB = 6
_NSS = 2
_NTURN = _NG // _NB
_CH = _NPAD // _NS
_NAGG = 10112
_CHA = _NAGG // _NS
_PB = 512


def _pre_body(x_ref, w_ref, b_ref, wm_ref, bm_ref, wa_ref,
              h_ref, g_ref, es1_ref, es2_ref):
    x = x_ref[...]
    h_raw = lax.dot_general(x, w_ref[...], (((1,), (1,)), ((), ())),
                            preferred_element_type=jnp.float32) + b_ref[...]
    nrm = jnp.sqrt(jnp.sum(h_raw * h_raw, axis=1, keepdims=True))
    h_dir = h_raw / jnp.maximum(nrm, 1e-12)
    mag_lin = jnp.sum(x * wm_ref[...], axis=1, keepdims=True) + bm_ref[...]
    mag = jax.nn.sigmoid(mag_lin) * 1.5
    h = h_dir * mag
    h_ref[...] = h
    wa = wa_ref[...]
    es1 = jnp.exp(jnp.sum(h * wa[0:1, :], axis=1, keepdims=True))
    es1_ref[...] = es1
    es2_ref[...] = jnp.exp(jnp.sum(h * wa[1:2, :], axis=1, keepdims=True))
    g_ref[...] = h * es1


_pre_call = pl.pallas_call(
    _pre_body,
    grid=(_NPAD // _PB,),
    in_specs=[
        pl.BlockSpec((_PB, _D), lambda i: (i, 0)),
        pl.BlockSpec((_D, _D), lambda i: (0, 0)),
        pl.BlockSpec((1, _D), lambda i: (0, 0)),
        pl.BlockSpec((1, _D), lambda i: (0, 0)),
        pl.BlockSpec((1, 1), lambda i: (0, 0)),
        pl.BlockSpec((2, _D), lambda i: (0, 0)),
    ],
    out_specs=[
        pl.BlockSpec((_PB, _D), lambda i: (i, 0)),
        pl.BlockSpec((_PB, _D), lambda i: (i, 0)),
        pl.BlockSpec((_PB, 1), lambda i: (i, 0)),
        pl.BlockSpec((_PB, 1), lambda i: (i, 0)),
    ],
    out_shape=[
        jax.ShapeDtypeStruct((_NPAD, _D), jnp.float32),
        jax.ShapeDtypeStruct((_NPAD, _D), jnp.float32),
        jax.ShapeDtypeStruct((_NPAD, 1), jnp.float32),
        jax.ShapeDtypeStruct((_NPAD, 1), jnp.float32),
    ],
)


def _post_body(h_ref, a0_ref, a1_ref, es2_ref, s10_ref, s11_ref, o_ref):
    es2 = es2_ref[...]
    f = es2 / (es2 * (s10_ref[...] + s11_ref[...]) + 1e-10)
    hc = h_ref[...] + f * (a0_ref[...] + a1_ref[...])
    un = jnp.sqrt(jnp.sum(hc * hc, axis=1, keepdims=True))
    unc = jnp.maximum(un, 1e-15)
    hp = jnp.tanh(unc) * hc / unc
    nn = jnp.sqrt(jnp.sum(hp * hp, axis=1, keepdims=True))
    o_ref[...] = jnp.where(nn > 0.95, hp * (0.95 / (nn + 1e-8)), hp)


_post_call = pl.pallas_call(
    _post_body,
    grid=(_NPAD // _PB,),
    in_specs=[pl.BlockSpec((_PB, _D), lambda i: (i, 0))] * 3
    + [pl.BlockSpec((_PB, 1), lambda i: (i, 0))] * 3,
    out_specs=pl.BlockSpec((_PB, _D), lambda i: (i, 0)),
    out_shape=jax.ShapeDtypeStruct((_NPAD, _D), jnp.float32),
)


_mesh = plsc.VectorSubcoreMesh(core_axis_name="c", subcore_axis_name="s")


@functools.partial(
    pl.kernel,
    out_type=jax.ShapeDtypeStruct((_NC, _NPAD), jnp.float32),
    mesh=_mesh,
    compiler_params=pltpu.CompilerParams(needs_layout_passes=False),
    scratch_types=[
        pltpu.VMEM((_NPAD,), jnp.float32),
        pltpu.VMEM((_EPW,), jnp.int32),
        pltpu.VMEM((_EPW,), jnp.int32),
        pltpu.VMEM((_NPAD,), jnp.float32),
        pltpu.VMEM((_NS, _CH), jnp.float32),
        pltpu.VMEM_SHARED((_NS, _NPAD), jnp.float32),
    ],
)
def _edge_phase_a(src_h, tgt_h, es1_h, sump_h,
                  es1_v, src_v, tgt_v, sum_v, red_v, shared):
    c = lax.axis_index("c")
    s = lax.axis_index("s")
    wid = s * _NC + c
    base = wid * _EPW
    pltpu.sync_copy(es1_h, es1_v)
    pltpu.sync_copy(src_h.at[pl.ds(base, _EPW)], src_v)
    pltpu.sync_copy(tgt_h.at[pl.ds(base, _EPW)], tgt_v)

    zv = jnp.zeros((16,), jnp.float32)

    def _zero(i, carry):
        sum_v[pl.ds(i * 16, 16)] = zv
        return carry

    lax.fori_loop(0, _NPAD // 16, _zero, 0)

    def _edges(i, carry):
        o = i * 16
        si = src_v[pl.ds(o, 16)]
        ti = tgt_v[pl.ds(o, 16)]
        plsc.addupdate_scatter(sum_v, [ti], plsc.load_gather(es1_v, [si]))
        return carry

    lax.fori_loop(0, _EPW // 16, _edges, 0)

    pltpu.sync_copy(sum_v, shared.at[s])
    plsc.subcore_barrier()

    ch = s * _CH
    pltpu.sync_copy(shared.at[:, pl.ds(ch, _CH)], red_v)

    def _reduce(j, carry):
        acc = red_v[0, pl.ds(j * 16, 16)]
        for k in range(1, _NS):
            acc = acc + red_v[k, pl.ds(j * 16, 16)]
        sum_v[pl.ds(j * 16, 16)] = acc
        return carry

    lax.fori_loop(0, _CH // 16, _reduce, 0)
    pltpu.sync_copy(sum_v.at[pl.ds(0, _CH)], sump_h.at[c, pl.ds(ch, _CH)])


@functools.partial(
    pl.kernel,
    out_type=jax.ShapeDtypeStruct((_NC, _NAGG, _D), jnp.float32),
    mesh=_mesh,
    compiler_params=pltpu.CompilerParams(needs_layout_passes=False),
    scratch_types=[
        pltpu.VMEM((_EPW,), jnp.int32),
        pltpu.VMEM((_EPW,), jnp.int32),
    ]
    + [pltpu.VMEM((_G, _D), jnp.float32)] * _NB
    + [pltpu.VMEM_SHARED((_NAGG, _D), jnp.float32)]
    + [pltpu.SemaphoreType.DMA] * (_NB + _NSS),
)
def _edge_phase_b(src_h, tgt_h, g_hbm, aggp_h,
                  src_v, tgt_v, *rest):
    bufs = rest[:_NB]
    agg_s = rest[_NB]
    sems = rest[_NB + 1:]
    gsem = sems[:_NB]
    ssem = sems[_NB:]
    c = lax.axis_index("c")
    s = lax.axis_index("s")
    wid = s * _NC + c
    base = wid * _EPW
    pltpu.sync_copy(src_h.at[pl.ds(base, _EPW)], src_v)
    pltpu.sync_copy(tgt_h.at[pl.ds(base, _EPW)], tgt_v)

    zv = jnp.zeros((16,), jnp.float32)
    for r in (4, 5):
        for j in range(_G):
            for k in range(_D // 16):
                bufs[r][j, pl.ds(k * 16, 16)] = zv

    rbase = s * _CHA
    _ZF = _CHA // _G
    _ZR = _CHA - _ZF * _G

    def _zero(k, carry):
        pltpu.sync_copy(bufs[4], agg_s.at[pl.ds(rbase + k * _G, _G), :])
        return carry

    lax.fori_loop(0, _ZF, _zero, 0)
    pltpu.sync_copy(bufs[4].at[pl.ds(0, _ZR), :],
                    agg_s.at[pl.ds(rbase + _ZF * _G, _ZR), :])
    plsc.subcore_barrier()

    dummy = jnp.full((16,), _N, jnp.int32)
    for r in range(4):
        si = src_v[pl.ds(r * _G, _G)]
        pltpu.async_copy(g_hbm.at[si], bufs[r], gsem[r])
    for q in range(_NSS):
        pltpu.async_copy(bufs[4 + q], agg_s.at[dummy], ssem[q], add=True)

    def _six_turns(k6, carry):
        for t in range(_NB):
            g = k6 * _NB + t
            o = g * _G
            siw = src_v[pl.ds(o, _G)]
            pltpu.make_async_copy(g_hbm.at[siw], bufs[t], gsem[t]).wait()
            pltpu.make_async_copy(bufs[t], agg_s.at[dummy],
                                  ssem[t % _NSS]).wait()
            ti = tgt_v[pl.ds(o, _G)]
            pltpu.async_copy(bufs[t], agg_s.at[ti], ssem[t % _NSS],
                             add=True)

            @pl.when(g + 4 < _NG)
            def _next_gather():
                si = src_v[pl.ds(o + 4 * _G, _G)]
                nb = (t + 4) % _NB
                pltpu.async_copy(g_hbm.at[si], bufs[nb], gsem[nb])
        return carry

    lax.fori_loop(0, _NTURN, _six_turns, 0)

    for q in range(_NSS):
        pltpu.make_async_copy(bufs[q], agg_s.at[dummy], ssem[q]).wait()
    plsc.subcore_barrier()

    def _out(k, carry):
        r = rbase + k * _G
        pltpu.sync_copy(agg_s.at[pl.ds(r, _G), :], bufs[0])
        pltpu.sync_copy(bufs[0], aggp_h.at[c, pl.ds(r, _G), :])
        return carry

    lax.fori_loop(0, _ZF, _out, 0)
    r_tail = rbase + _ZF * _G
    pltpu.sync_copy(agg_s.at[pl.ds(r_tail, _ZR), :],
                    bufs[0].at[pl.ds(0, _ZR), :])
    pltpu.sync_copy(bufs[0].at[pl.ds(0, _ZR), :],
                    aggp_h.at[c, pl.ds(r_tail, _ZR), :])


def kernel(x, edge_index, W, b, Wm, bm, Wa):
    x_p = jnp.pad(x, ((0, _NPAD - _N), (0, 0)))
    h, g, es1c, es2c = _pre_call(x_p, W, b.reshape(1, _D), Wm,
                                 bm.reshape(1, 1), Wa.reshape(2, _D))
    pad_e = _EPAD - _E
    src = jnp.concatenate([edge_index[0], jnp.zeros((pad_e,), jnp.int32)])
    tgt = jnp.concatenate([edge_index[1], jnp.full((pad_e,), _N, jnp.int32)])
    es1 = es1c.reshape(_NPAD)
    sump = _edge_phase_a(src, tgt, es1)
    aggp = _edge_phase_b(src, tgt, g)
    agg0 = jnp.pad(aggp[0], ((0, _NPAD - _NAGG), (0, 0)))
    agg1 = jnp.pad(aggp[1], ((0, _NPAD - _NAGG), (0, 0)))
    s10 = sump[0].reshape(_NPAD, 1)
    s11 = sump[1].reshape(_NPAD, 1)
    out = _post_call(h, agg0, agg1, es2c, s10, s11)
    return out[:_N]
